# Initial kernel scaffold; baseline (speedup 1.0000x reference)
import jax
import jax.numpy as jnp
from jax.experimental import pallas as pl


def kernel(x, edge_index, batch, W1, b1, W2, att_src, att_dst, b2, fc1_w, fc1_b, fc2_w, fc2_b):
    raise NotImplementedError("write your pallas kernel here")


# speedup vs baseline: 35.4156x; 35.4156x over previous
import functools

import jax
import jax.numpy as jnp
from jax import lax
from jax.experimental import pallas as pl
from jax.experimental.pallas import tpu as pltpu
from jax.experimental.pallas import tpu_sc as plsc

N = 10000
E = 160000
F_IN = 256
HID = 16
NH = 8
HD = NH * HID
G = 64
ACCW = HD + 16

NC = 2
NS = 16
NW = NC * NS
EB = 128
EP = 172032
EPW = EP // NW
NB = EPW // EB
NROWS = N + 112
RPT = NROWS // NS

_mesh = plsc.VectorSubcoreMesh(core_axis_name="c", subcore_axis_name="s")
_sc_params = pltpu.CompilerParams(use_tc_tiling_on_sc=False,
                                  needs_layout_passes=False)


def _wid_rs():
    c = lax.axis_index("c")
    s = lax.axis_index("s")
    return c, s, c * NS + s, s * RPT


def _deg_body(dst_h, ones_h, z_h, out_h, idx_v, ones_v, acc, sem):
    c, s, wid, rs = _wid_rs()
    pltpu.sync_copy(z_h.at[pl.ds(rs, RPT)], acc.at[pl.ds(rs, RPT)])
    pltpu.sync_copy(ones_h, ones_v)
    plsc.subcore_barrier()

    def step(j, carry):
        base = wid * EPW + j * EB
        pltpu.sync_copy(dst_h.at[pl.ds(base, EB)], idx_v)
        pltpu.sync_copy(ones_v, acc.at[idx_v], add=True)
        return carry

    lax.fori_loop(0, NB, step, 0)
    plsc.subcore_barrier()
    pltpu.sync_copy(acc.at[pl.ds(rs, RPT)], out_h.at[c, pl.ds(rs, RPT)])


_deg_call = pl.kernel(
    _deg_body,
    out_type=jax.ShapeDtypeStruct((NC, NROWS, HID), jnp.float32),
    mesh=_mesh,
    compiler_params=_sc_params,
    scratch_types=[
        pltpu.VMEM((EB,), jnp.int32),
        pltpu.VMEM((EB, HID), jnp.float32),
        pltpu.VMEM_SHARED((NROWS, HID), jnp.float32),
        pltpu.SemaphoreType.DMA,
    ],
)


def _gcn_body(src_h, dst_h, g_h, z_h, out_h, sidx, didx, rows_v, acc, sem):
    c, s, wid, rs = _wid_rs()
    pltpu.sync_copy(z_h.at[pl.ds(rs, RPT)], acc.at[pl.ds(rs, RPT)])
    plsc.subcore_barrier()

    def step(j, carry):
        base = wid * EPW + j * EB
        pltpu.sync_copy(src_h.at[pl.ds(base, EB)], sidx)
        pltpu.sync_copy(dst_h.at[pl.ds(base, EB)], didx)
        pltpu.async_copy(g_h.at[sidx], rows_v, sem).wait()
        pltpu.sync_copy(rows_v, acc.at[didx], add=True)
        return carry

    lax.fori_loop(0, NB, step, 0)
    plsc.subcore_barrier()
    pltpu.sync_copy(acc.at[pl.ds(rs, RPT)], out_h.at[c, pl.ds(rs, RPT)])


_gcn_call = pl.kernel(
    _gcn_body,
    out_type=jax.ShapeDtypeStruct((NC, NROWS, HID), jnp.float32),
    mesh=_mesh,
    compiler_params=_sc_params,
    scratch_types=[
        pltpu.VMEM((EB,), jnp.int32),
        pltpu.VMEM((EB,), jnp.int32),
        pltpu.VMEM((EB, HID), jnp.float32),
        pltpu.VMEM_SHARED((NROWS, HID), jnp.float32),
        pltpu.SemaphoreType.DMA,
    ],
)


def _gat_body(src_h, dst_h, hs_h, ad_h, z_h, out_h,
              sidx, didx, srows, drows, acc, sem):
    c, s, wid, rs = _wid_rs()
    pltpu.sync_copy(z_h.at[pl.ds(rs, RPT)], acc.at[pl.ds(rs, RPT)])
    plsc.subcore_barrier()

    def step(j, carry):
        base = wid * EPW + j * EB
        pltpu.sync_copy(src_h.at[pl.ds(base, EB)], sidx)
        pltpu.sync_copy(dst_h.at[pl.ds(base, EB)], didx)
        cp1 = pltpu.async_copy(hs_h.at[sidx], srows, sem)
        cp2 = pltpu.async_copy(ad_h.at[didx], drows, sem)
        cp1.wait()
        cp2.wait()

        lane = lax.iota(jnp.int32, 16)

        def edge(i, ecarry):
            e = srows[i, pl.ds(HD, 16)] + drows[i, pl.ds(0, 16)]
            w = jnp.exp(jnp.where(e >= 0.0, e, 0.2 * e))
            for h in range(NH):
                wh = jnp.sum(jnp.where(lane == h, w, 0.0))
                srows[i, pl.ds(h * HID, HID)] = (
                    srows[i, pl.ds(h * HID, HID)] * wh)
            srows[i, pl.ds(HD, 16)] = w
            return ecarry

        lax.fori_loop(0, EB, edge, 0)
        pltpu.sync_copy(srows, acc.at[didx], add=True)
        return carry

    lax.fori_loop(0, NB, step, 0)
    plsc.subcore_barrier()
    pltpu.sync_copy(acc.at[pl.ds(rs, RPT)], out_h.at[c, pl.ds(rs, RPT)])


_gat_call = pl.kernel(
    _gat_body,
    out_type=jax.ShapeDtypeStruct((NC, NROWS, ACCW), jnp.float32),
    mesh=_mesh,
    compiler_params=_sc_params,
    scratch_types=[
        pltpu.VMEM((EB,), jnp.int32),
        pltpu.VMEM((EB,), jnp.int32),
        pltpu.VMEM((EB, ACCW), jnp.float32),
        pltpu.VMEM((EB, 16), jnp.float32),
        pltpu.VMEM_SHARED((NROWS, ACCW), jnp.float32),
        pltpu.SemaphoreType.DMA,
    ],
)


def _tc_b_body(degp, x, W1, g_ref, dinv_ref):
    deg = degp[0, :, 0:1] + degp[1, :, 0:1]
    dinv = jnp.where(deg > 0.0, lax.rsqrt(deg), 0.0)
    h = jnp.dot(x[...], W1[...], preferred_element_type=jnp.float32, precision=lax.Precision.HIGHEST)
    g_ref[...] = h * dinv
    dinv_ref[...] = dinv


def _tc_d_body(accp, dinv, b1, W2, asT, adT, x1_ref, hs_ref, ad_ref):
    acc = accp[0] + accp[1]
    blk = acc.shape[0]
    x1 = jnp.maximum(acc * dinv[...] + b1[...], 0.0)
    h2 = jnp.dot(x1, W2[...], preferred_element_type=jnp.float32, precision=lax.Precision.HIGHEST)
    a_s = [jnp.dot(W2[:, h * HID:(h + 1) * HID], asT[h],
                   preferred_element_type=jnp.float32, precision=lax.Precision.HIGHEST) for h in range(NH)]
    a_d = [jnp.dot(W2[:, h * HID:(h + 1) * HID], adT[h],
                   preferred_element_type=jnp.float32, precision=lax.Precision.HIGHEST) for h in range(NH)]
    A = jnp.concatenate(a_s + a_d, axis=1)
    proj = jnp.dot(x1, A, preferred_element_type=jnp.float32, precision=lax.Precision.HIGHEST)
    x1_ref[...] = x1
    hs_ref[:, :HD] = h2
    hs_ref[:, HD:HD + 8] = proj[:, :8]
    hs_ref[:, HD + 8:] = jnp.zeros((blk, 8), jnp.float32)
    ad_ref[:, :8] = proj[:, 8:]
    ad_ref[:, 8:] = jnp.zeros((blk, 8), jnp.float32)


def _tc_f_body(accp, x1, b2, batch2, fc1_w, fc1_b, fc2_w, fc2_b, out_ref):
    acc = accp[0, :N, :] + accp[1, :N, :]
    num = acc[:, :HD]
    den8 = acc[:, HD:HD + 8]
    hrow = lax.broadcasted_iota(jnp.int32, (8, HD), 0)
    hcol = lax.div(lax.broadcasted_iota(jnp.int32, (8, HD), 1),
                   jnp.full((8, HD), HID, jnp.int32))
    expand = (hrow == hcol).astype(jnp.float32)
    den = jnp.dot(den8, expand, preferred_element_type=jnp.float32, precision=lax.Precision.HIGHEST)
    x2 = jnp.maximum(num / (den + 1e-16) + b2[...], 0.0)
    gid = lax.broadcasted_iota(jnp.int32, (G, N), 0)
    mask = (gid == batch2[...]).astype(jnp.float32)
    sums1 = jnp.dot(mask, x1[...], preferred_element_type=jnp.float32, precision=lax.Precision.HIGHEST)
    sums2 = jnp.dot(mask, x2, preferred_element_type=jnp.float32, precision=lax.Precision.HIGHEST)
    cnt = jnp.sum(mask, axis=1, keepdims=True)
    inv = 1.0 / jnp.maximum(cnt, 1.0)
    hfc = jnp.maximum(
        jnp.dot(sums1 * inv, fc1_w[:HID, :], preferred_element_type=jnp.float32, precision=lax.Precision.HIGHEST)
        + jnp.dot(sums2 * inv, fc1_w[HID:, :], preferred_element_type=jnp.float32, precision=lax.Precision.HIGHEST)
        + fc1_b[...], 0.0)
    out_ref[...] = (jnp.dot(hfc, fc2_w[...], preferred_element_type=jnp.float32, precision=lax.Precision.HIGHEST)
                    + fc2_b[...])


def _tc_b(degp, x, W1):
    nblk = 10
    rb = N // nblk
    return pl.pallas_call(
        _tc_b_body,
        grid=(nblk,),
        in_specs=[
            pl.BlockSpec((NC, rb, HID), lambda i: (0, i, 0)),
            pl.BlockSpec((rb, F_IN), lambda i: (i, 0)),
            pl.BlockSpec((F_IN, HID), lambda i: (0, 0)),
        ],
        out_specs=[
            pl.BlockSpec((rb, HID), lambda i: (i, 0)),
            pl.BlockSpec((rb, 1), lambda i: (i, 0)),
        ],
        out_shape=[jax.ShapeDtypeStruct((N, HID), jnp.float32),
                   jax.ShapeDtypeStruct((N, 1), jnp.float32)],
    )(degp, x, W1)


def _tc_d(accp, dinv, b1, W2, asT, adT):
    nblk = 10
    rb = N // nblk
    return pl.pallas_call(
        _tc_d_body,
        grid=(nblk,),
        in_specs=[
            pl.BlockSpec((NC, rb, HID), lambda i: (0, i, 0)),
            pl.BlockSpec((rb, 1), lambda i: (i, 0)),
            pl.BlockSpec((1, HID), lambda i: (0, 0)),
            pl.BlockSpec((HID, HD), lambda i: (0, 0)),
            pl.BlockSpec((NH, HID, 1), lambda i: (0, 0, 0)),
            pl.BlockSpec((NH, HID, 1), lambda i: (0, 0, 0)),
        ],
        out_specs=[
            pl.BlockSpec((rb, HID), lambda i: (i, 0)),
            pl.BlockSpec((rb, ACCW), lambda i: (i, 0)),
            pl.BlockSpec((rb, 16), lambda i: (i, 0)),
        ],
        out_shape=[jax.ShapeDtypeStruct((N, HID), jnp.float32),
                   jax.ShapeDtypeStruct((N, ACCW), jnp.float32),
                   jax.ShapeDtypeStruct((N, 16), jnp.float32)],
    )(accp, dinv, b1, W2, asT, adT)


def _tc_f(accp, x1, b2, batch2, fc1_w, fc1_b, fc2_w, fc2_b):
    return pl.pallas_call(
        _tc_f_body,
        out_shape=jax.ShapeDtypeStruct((G, 1), jnp.float32),
    )(accp, x1, b2, batch2, fc1_w, fc1_b, fc2_w, fc2_b)


def kernel(x, edge_index, batch, W1, b1, W2, att_src, att_dst, b2,
           fc1_w, fc1_b, fc2_w, fc2_b):
    loop = jnp.arange(N, dtype=jnp.int32)
    src = jnp.concatenate([edge_index[0], loop])
    dst = jnp.concatenate([edge_index[1], loop])
    pad = EP - (E + N)
    src_p = jnp.concatenate([src, jnp.zeros((pad,), jnp.int32)])
    dst_p = jnp.concatenate([dst, jnp.full((pad,), N, jnp.int32)])

    ones128 = jnp.ones((EB, HID), jnp.float32)
    z16 = jnp.zeros((NROWS, HID), jnp.float32)
    z144 = jnp.zeros((NROWS, ACCW), jnp.float32)

    degp = _deg_call(dst_p, ones128, z16)
    g, dinv = _tc_b(degp, x, W1)
    accp = _gcn_call(src_p, dst_p, g, z16)
    asT = att_src.reshape(NH, HID, 1)
    adT = att_dst.reshape(NH, HID, 1)
    x1, hs_t, ad_t = _tc_d(accp, dinv, b1.reshape(1, HID), W2, asT, adT)
    accp2 = _gat_call(src_p, dst_p, hs_t, ad_t, z144)
    out = _tc_f(accp2, x1, b2.reshape(1, HD), batch.reshape(1, N),
                fc1_w, fc1_b.reshape(1, HID), fc2_w, fc2_b.reshape(1, 1))
    return out.reshape(-1)


# speedup vs baseline: 43.9898x; 1.2421x over previous
import functools

import jax
import jax.numpy as jnp
from jax import lax
from jax.experimental import pallas as pl
from jax.experimental.pallas import tpu as pltpu
from jax.experimental.pallas import tpu_sc as plsc

N = 10000
E = 160000
F_IN = 256
HID = 16
NH = 8
HD = NH * HID
G = 64
ACCW = HD + 16

NC = 2
NS = 16
NW = NC * NS
EB = 128
EP = 172032
EPW = EP // NW
NB = EPW // EB
NROWS = N + 112
RPT = NROWS // NS

_mesh = plsc.VectorSubcoreMesh(core_axis_name="c", subcore_axis_name="s")
_sc_params = pltpu.CompilerParams(use_tc_tiling_on_sc=False,
                                  needs_layout_passes=False)


def _wid_rs():
    c = lax.axis_index("c")
    s = lax.axis_index("s")
    return c, s, c * NS + s, s * RPT


def _deg_body(dst_h, ones_h, z_h, out_h, idx_v, ones_v, acc, sem):
    c, s, wid, rs = _wid_rs()
    pltpu.sync_copy(z_h.at[pl.ds(rs, RPT)], acc.at[pl.ds(rs, RPT)])
    pltpu.sync_copy(ones_h, ones_v)
    plsc.subcore_barrier()

    def step(j, carry):
        base = wid * EPW + j * EB
        pltpu.sync_copy(dst_h.at[pl.ds(base, EB)], idx_v)
        pltpu.sync_copy(ones_v, acc.at[idx_v], add=True)
        return carry

    lax.fori_loop(0, NB, step, 0)
    plsc.subcore_barrier()
    pltpu.sync_copy(acc.at[pl.ds(rs, RPT)], out_h.at[c, pl.ds(rs, RPT)])


_deg_call = pl.kernel(
    _deg_body,
    out_type=jax.ShapeDtypeStruct((NC, NROWS, HID), jnp.float32),
    mesh=_mesh,
    compiler_params=_sc_params,
    scratch_types=[
        pltpu.VMEM((EB,), jnp.int32),
        pltpu.VMEM((EB, HID), jnp.float32),
        pltpu.VMEM_SHARED((NROWS, HID), jnp.float32),
        pltpu.SemaphoreType.DMA,
    ],
)


def _gcn_body(src_h, dst_h, g_h, z_h, out_h, sidx, didx, rows_v, acc, sem):
    c, s, wid, rs = _wid_rs()
    pltpu.sync_copy(z_h.at[pl.ds(rs, RPT)], acc.at[pl.ds(rs, RPT)])
    plsc.subcore_barrier()

    def step(j, carry):
        base = wid * EPW + j * EB
        pltpu.sync_copy(src_h.at[pl.ds(base, EB)], sidx)
        pltpu.sync_copy(dst_h.at[pl.ds(base, EB)], didx)
        pltpu.async_copy(g_h.at[sidx], rows_v, sem).wait()
        pltpu.sync_copy(rows_v, acc.at[didx], add=True)
        return carry

    lax.fori_loop(0, NB, step, 0)
    plsc.subcore_barrier()
    pltpu.sync_copy(acc.at[pl.ds(rs, RPT)], out_h.at[c, pl.ds(rs, RPT)])


_gcn_call = pl.kernel(
    _gcn_body,
    out_type=jax.ShapeDtypeStruct((NC, NROWS, HID), jnp.float32),
    mesh=_mesh,
    compiler_params=_sc_params,
    scratch_types=[
        pltpu.VMEM((EB,), jnp.int32),
        pltpu.VMEM((EB,), jnp.int32),
        pltpu.VMEM((EB, HID), jnp.float32),
        pltpu.VMEM_SHARED((NROWS, HID), jnp.float32),
        pltpu.SemaphoreType.DMA,
    ],
)


def _gat_body(src_h, dst_h, hs_h, ad_h, z_h, out_h,
              sidx, didx, srows, drows, acc, sem):
    c, s, wid, rs = _wid_rs()
    pltpu.sync_copy(z_h.at[pl.ds(rs, RPT)], acc.at[pl.ds(rs, RPT)])
    plsc.subcore_barrier()

    def step(j, carry):
        base = wid * EPW + j * EB
        pltpu.sync_copy(src_h.at[pl.ds(base, EB)], sidx)
        pltpu.sync_copy(dst_h.at[pl.ds(base, EB)], didx)
        cp1 = pltpu.async_copy(hs_h.at[sidx], srows, sem)
        cp2 = pltpu.async_copy(ad_h.at[didx], drows, sem)
        cp1.wait()
        cp2.wait()

        lane = lax.iota(jnp.int32, 16)

        @plsc.parallel_loop(0, EB, unroll=4)
        def edge(i):
            e = srows[i, pl.ds(HD, 16)] + drows[i, pl.ds(0, 16)]
            w = jnp.exp(jnp.where(e >= 0.0, e, 0.2 * e))
            for h in range(NH):
                wh = jnp.sum(jnp.where(lane == h, w, 0.0))
                srows[i, pl.ds(h * HID, HID)] = (
                    srows[i, pl.ds(h * HID, HID)] * wh)
            srows[i, pl.ds(HD, 16)] = w
        pltpu.sync_copy(srows, acc.at[didx], add=True)
        return carry

    lax.fori_loop(0, NB, step, 0)
    plsc.subcore_barrier()
    pltpu.sync_copy(acc.at[pl.ds(rs, RPT)], out_h.at[c, pl.ds(rs, RPT)])


_gat_call = pl.kernel(
    _gat_body,
    out_type=jax.ShapeDtypeStruct((NC, NROWS, ACCW), jnp.float32),
    mesh=_mesh,
    compiler_params=_sc_params,
    scratch_types=[
        pltpu.VMEM((EB,), jnp.int32),
        pltpu.VMEM((EB,), jnp.int32),
        pltpu.VMEM((EB, ACCW), jnp.float32),
        pltpu.VMEM((EB, 16), jnp.float32),
        pltpu.VMEM_SHARED((NROWS, ACCW), jnp.float32),
        pltpu.SemaphoreType.DMA,
    ],
)


def _tc_b_body(degp, x, W1, g_ref, dinv_ref):
    deg = degp[0, :, 0:1] + degp[1, :, 0:1]
    dinv = jnp.where(deg > 0.0, lax.rsqrt(deg), 0.0)
    h = jnp.dot(x[...], W1[...], preferred_element_type=jnp.float32, precision=lax.Precision.HIGHEST)
    g_ref[...] = h * dinv
    dinv_ref[...] = dinv


def _tc_d_body(accp, dinv, b1, W2, asT, adT, x1_ref, hs_ref, ad_ref):
    acc = accp[0] + accp[1]
    blk = acc.shape[0]
    x1 = jnp.maximum(acc * dinv[...] + b1[...], 0.0)
    h2 = jnp.dot(x1, W2[...], preferred_element_type=jnp.float32, precision=lax.Precision.HIGHEST)
    a_s = [jnp.dot(W2[:, h * HID:(h + 1) * HID], asT[h],
                   preferred_element_type=jnp.float32, precision=lax.Precision.HIGHEST) for h in range(NH)]
    a_d = [jnp.dot(W2[:, h * HID:(h + 1) * HID], adT[h],
                   preferred_element_type=jnp.float32, precision=lax.Precision.HIGHEST) for h in range(NH)]
    A = jnp.concatenate(a_s + a_d, axis=1)
    proj = jnp.dot(x1, A, preferred_element_type=jnp.float32, precision=lax.Precision.HIGHEST)
    x1_ref[...] = x1
    hs_ref[:, :HD] = h2
    hs_ref[:, HD:HD + 8] = proj[:, :8]
    hs_ref[:, HD + 8:] = jnp.zeros((blk, 8), jnp.float32)
    ad_ref[:, :8] = proj[:, 8:]
    ad_ref[:, 8:] = jnp.zeros((blk, 8), jnp.float32)


def _tc_f_body(accp, x1, b2, batch2, fc1_w, fc1_b, fc2_w, fc2_b, out_ref):
    acc = accp[0, :N, :] + accp[1, :N, :]
    num = acc[:, :HD]
    den8 = acc[:, HD:HD + 8]
    hrow = lax.broadcasted_iota(jnp.int32, (8, HD), 0)
    hcol = lax.div(lax.broadcasted_iota(jnp.int32, (8, HD), 1),
                   jnp.full((8, HD), HID, jnp.int32))
    expand = (hrow == hcol).astype(jnp.float32)
    den = jnp.dot(den8, expand, preferred_element_type=jnp.float32, precision=lax.Precision.HIGHEST)
    x2 = jnp.maximum(num / (den + 1e-16) + b2[...], 0.0)
    gid = lax.broadcasted_iota(jnp.int32, (G, N), 0)
    mask = (gid == batch2[...]).astype(jnp.float32)
    sums1 = jnp.dot(mask, x1[...], preferred_element_type=jnp.float32, precision=lax.Precision.HIGHEST)
    sums2 = jnp.dot(mask, x2, preferred_element_type=jnp.float32, precision=lax.Precision.HIGHEST)
    cnt = jnp.sum(mask, axis=1, keepdims=True)
    inv = 1.0 / jnp.maximum(cnt, 1.0)
    hfc = jnp.maximum(
        jnp.dot(sums1 * inv, fc1_w[:HID, :], preferred_element_type=jnp.float32, precision=lax.Precision.HIGHEST)
        + jnp.dot(sums2 * inv, fc1_w[HID:, :], preferred_element_type=jnp.float32, precision=lax.Precision.HIGHEST)
        + fc1_b[...], 0.0)
    out_ref[...] = (jnp.dot(hfc, fc2_w[...], preferred_element_type=jnp.float32, precision=lax.Precision.HIGHEST)
                    + fc2_b[...])


def _tc_b(degp, x, W1):
    nblk = 10
    rb = N // nblk
    return pl.pallas_call(
        _tc_b_body,
        grid=(nblk,),
        in_specs=[
            pl.BlockSpec((NC, rb, HID), lambda i: (0, i, 0)),
            pl.BlockSpec((rb, F_IN), lambda i: (i, 0)),
            pl.BlockSpec((F_IN, HID), lambda i: (0, 0)),
        ],
        out_specs=[
            pl.BlockSpec((rb, HID), lambda i: (i, 0)),
            pl.BlockSpec((rb, 1), lambda i: (i, 0)),
        ],
        out_shape=[jax.ShapeDtypeStruct((N, HID), jnp.float32),
                   jax.ShapeDtypeStruct((N, 1), jnp.float32)],
    )(degp, x, W1)


def _tc_d(accp, dinv, b1, W2, asT, adT):
    nblk = 10
    rb = N // nblk
    return pl.pallas_call(
        _tc_d_body,
        grid=(nblk,),
        in_specs=[
            pl.BlockSpec((NC, rb, HID), lambda i: (0, i, 0)),
            pl.BlockSpec((rb, 1), lambda i: (i, 0)),
            pl.BlockSpec((1, HID), lambda i: (0, 0)),
            pl.BlockSpec((HID, HD), lambda i: (0, 0)),
            pl.BlockSpec((NH, HID, 1), lambda i: (0, 0, 0)),
            pl.BlockSpec((NH, HID, 1), lambda i: (0, 0, 0)),
        ],
        out_specs=[
            pl.BlockSpec((rb, HID), lambda i: (i, 0)),
            pl.BlockSpec((rb, ACCW), lambda i: (i, 0)),
            pl.BlockSpec((rb, 16), lambda i: (i, 0)),
        ],
        out_shape=[jax.ShapeDtypeStruct((N, HID), jnp.float32),
                   jax.ShapeDtypeStruct((N, ACCW), jnp.float32),
                   jax.ShapeDtypeStruct((N, 16), jnp.float32)],
    )(accp, dinv, b1, W2, asT, adT)


def _tc_f(accp, x1, b2, batch2, fc1_w, fc1_b, fc2_w, fc2_b):
    return pl.pallas_call(
        _tc_f_body,
        out_shape=jax.ShapeDtypeStruct((G, 1), jnp.float32),
    )(accp, x1, b2, batch2, fc1_w, fc1_b, fc2_w, fc2_b)


def kernel(x, edge_index, batch, W1, b1, W2, att_src, att_dst, b2,
           fc1_w, fc1_b, fc2_w, fc2_b):
    loop = jnp.arange(N, dtype=jnp.int32)
    src = jnp.concatenate([edge_index[0], loop])
    dst = jnp.concatenate([edge_index[1], loop])
    pad = EP - (E + N)
    src_p = jnp.concatenate([src, jnp.zeros((pad,), jnp.int32)])
    dst_p = jnp.concatenate([dst, jnp.full((pad,), N, jnp.int32)])

    ones128 = jnp.ones((EB, HID), jnp.float32)
    z16 = jnp.zeros((NROWS, HID), jnp.float32)
    z144 = jnp.zeros((NROWS, ACCW), jnp.float32)

    degp = _deg_call(dst_p, ones128, z16)
    g, dinv = _tc_b(degp, x, W1)
    accp = _gcn_call(src_p, dst_p, g, z16)
    asT = att_src.reshape(NH, HID, 1)
    adT = att_dst.reshape(NH, HID, 1)
    x1, hs_t, ad_t = _tc_d(accp, dinv, b1.reshape(1, HID), W2, asT, adT)
    accp2 = _gat_call(src_p, dst_p, hs_t, ad_t, z144)
    out = _tc_f(accp2, x1, b2.reshape(1, HD), batch.reshape(1, N),
                fc1_w, fc1_b.reshape(1, HID), fc2_w, fc2_b.reshape(1, 1))
    return out.reshape(-1)
